# baseline (speedup 1.0000x reference)
import functools

import jax
import jax.numpy as jnp
from jax import lax
from jax.experimental import pallas as pl
from jax.experimental.pallas import tpu as pltpu
from jax.experimental.pallas import tpu_sc as plsc


def _sel_onehots(d):
    r = lax.broadcasted_iota(jnp.int32, (d, d // 2), 0)
    k = lax.broadcasted_iota(jnp.int32, (d, d // 2), 1)
    tgt = 32 * (k // 16) + (k % 16)
    sel_a = (r == tgt).astype(jnp.float32)
    sel_b = (r == tgt + 16).astype(jnp.float32)
    return sel_a, sel_b


def _edge_proj_body(eattr_ref, we_ref, be_ref, wc1_ref, bc1_ref, wc2_ref,
                    bc2_ref, o1_ref, o2_ref):
    eb = eattr_ref[...]
    sel_a, sel_b = _sel_onehots(we_ref.shape[0])
    we = we_ref[...]
    be = be_ref[...]
    for wc_ref, bc_ref, o_ref in ((wc1_ref, bc1_ref, o1_ref),
                                  (wc2_ref, bc2_ref, o2_ref)):
        wf = jnp.dot(we, wc_ref[...], preferred_element_type=jnp.float32)
        bf = jnp.dot(be, wc_ref[...],
                     preferred_element_type=jnp.float32) + bc_ref[...]
        wfa = jnp.dot(wf, sel_a, preferred_element_type=jnp.float32)
        wfb = jnp.dot(wf, sel_b, preferred_element_type=jnp.float32)
        bfa = jnp.dot(bf, sel_a, preferred_element_type=jnp.float32)
        bfb = jnp.dot(bf, sel_b, preferred_element_type=jnp.float32)
        a = jnp.dot(eb, wfa, preferred_element_type=jnp.float32) + bfa
        b = jnp.dot(eb, wfb, preferred_element_type=jnp.float32) + bfb
        au = lax.bitcast_convert_type(a, jnp.uint32)
        bu = lax.bitcast_convert_type(b, jnp.uint32)
        lo = (au + jnp.uint32(0x8000)) >> jnp.uint32(16)
        hi = (bu + jnp.uint32(0x8000)) & jnp.uint32(0xFFFF0000)
        o_ref[...] = lo | hi


def _edge_proj(edge_attr, W_e, b_e, W_c1, b_c1, W_c2, b_c2):
    E, D = edge_attr.shape
    H = W_e.shape[1]
    BE = 4000
    assert E % BE == 0
    grid = (E // BE,)
    full = lambda shape: pl.BlockSpec(shape, lambda i: (0, 0))
    return pl.pallas_call(
        _edge_proj_body,
        grid=grid,
        in_specs=[
            pl.BlockSpec((BE, D), lambda i: (i, 0)),
            full((D, H)), full((1, H)),
            full((H, D)), full((1, D)),
            full((H, D)), full((1, D)),
        ],
        out_specs=[
            pl.BlockSpec((BE, D // 2), lambda i: (i, 0)),
            pl.BlockSpec((BE, D // 2), lambda i: (i, 0)),
        ],
        out_shape=[
            jax.ShapeDtypeStruct((E, D // 2), jnp.uint32),
            jax.ShapeDtypeStruct((E, D // 2), jnp.uint32),
        ],
    )(edge_attr, W_e, b_e.reshape(1, -1), W_c1, b_c1.reshape(1, -1),
      W_c2, b_c2.reshape(1, -1))


_CK = 40
_NCH = 250


def _sc_agg_body(feat_hbm, ea_hbm, ids_hbm, zeros_hbm, out_hbm,
                 ids, rows, eav, accum,
                 si0, si1, si2, si3, si4, si5, si6, si7,
                 sg0, sg1, sg2, sg3, sh0, sh1, sh2, sh3,
                 se0, se1, se2, se3, ss0, ss1, ss2, ss3):
    nc = 2
    ns = 16
    c = lax.axis_index("c")
    s = lax.axis_index("s")
    wid = s * nc + c
    si = (si0, si1, si2, si3, si4, si5, si6, si7)
    sg = (sg0, sg1, sg2, sg3)
    sh = (sh0, sh1, sh2, sh3)
    se = (se0, se1, se2, se3)
    ss = (ss0, ss1, ss2, ss3)
    nch = _NCH
    ck = _CK
    estart = wid * nch * ck
    cstart = wid * nch

    @pl.when(s == 0)
    def _():
        pltpu.sync_copy(zeros_hbm, accum)

    plsc.subcore_barrier()

    def fetch_ids(j, b):
        pltpu.async_copy(ids_hbm.at[cstart + j], ids.at[b], si[b])

    def wait_ids(b):
        pltpu.make_async_copy(ids_hbm.at[0], ids.at[b], si[b]).wait()

    hk = ck // 2

    def fetch_rows(j, b, ib):
        pltpu.async_copy(ea_hbm.at[pl.ds(estart + j * ck, ck)], eav.at[b],
                         se[b])
        pltpu.async_copy(feat_hbm.at[ids.at[ib, 0, pl.ds(0, hk)]],
                         rows.at[b, pl.ds(0, hk)], sg[b])
        pltpu.async_copy(feat_hbm.at[ids.at[ib, 0, pl.ds(hk, hk)]],
                         rows.at[b, pl.ds(hk, hk)], sh[b])

    def wait_rows(b):
        pltpu.make_async_copy(ea_hbm.at[pl.ds(0, ck)], eav.at[b],
                              se[b]).wait()
        pltpu.make_async_copy(feat_hbm.at[ids.at[0, 0, pl.ds(0, hk)]],
                              rows.at[b, pl.ds(0, hk)], sg[b]).wait()
        pltpu.make_async_copy(feat_hbm.at[ids.at[0, 0, pl.ds(hk, hk)]],
                              rows.at[b, pl.ds(hk, hk)], sh[b]).wait()

    def scatter(b, ib):
        pltpu.async_copy(rows.at[b], accum.at[ids.at[ib, 1]], ss[b], add=True)

    def wait_scatter(b):
        pltpu.make_async_copy(rows.at[b], accum.at[ids.at[0, 1]],
                              ss[b]).wait()

    def compute(b):
        shp = jnp.full((16,), 16, jnp.uint32)
        msk = jnp.full((16,), 0xFFFF0000, jnp.uint32)

        def row(r, carry):
            for g in range(4):
                ew = eav[b, r, pl.ds(g * 16, 16)]
                ea = lax.bitcast_convert_type(ew << shp, jnp.float32)
                eb = lax.bitcast_convert_type(ew & msk, jnp.float32)
                slo = pl.ds(g * 32, 16)
                shi = pl.ds(g * 32 + 16, 16)
                rows[b, r, slo] = jnp.maximum(rows[b, r, slo] + ea, 0.0)
                rows[b, r, shi] = jnp.maximum(rows[b, r, shi] + eb, 0.0)
            return carry

        lax.fori_loop(0, ck, row, 0)

    def step(j, b, ib):
        @pl.when(j >= 2)
        def _():
            wait_scatter((b + 2) % 4)

        @pl.when(j + 3 < nch)
        def _():
            fetch_ids(j + 3, (ib + 3) % 8)

        @pl.when(j + 2 < nch)
        def _():
            wait_ids((ib + 2) % 8)
            fetch_rows(j + 2, (b + 2) % 4, (ib + 2) % 8)

        wait_rows(b)
        compute(b)
        scatter(b, ib)

    fetch_ids(0, 0)
    fetch_ids(1, 1)
    fetch_ids(2, 2)
    wait_ids(0)
    fetch_rows(0, 0, 0)
    wait_ids(1)
    fetch_rows(1, 1, 1)

    def octet(t, carry):
        for u in range(8):
            step(8 * t + u, u % 4, u)
        return carry

    lax.fori_loop(0, nch // 8, octet, 0)
    for e in range(nch - nch % 8, nch):
        step(e, e % 4, e % 8)
    wait_scatter((nch - 2) % 4)
    wait_scatter((nch - 1) % 4)
    plsc.subcore_barrier()

    @pl.when(s == 0)
    def _():
        pltpu.sync_copy(accum, out_hbm.at[c])


def _sc_aggregate(feat, ea, ids2, zeros):
    n, d = feat.shape
    mesh = plsc.VectorSubcoreMesh(core_axis_name="c", subcore_axis_name="s")
    kern = functools.partial(
        pl.kernel,
        mesh=mesh,
        out_type=jax.ShapeDtypeStruct((2, n, d), jnp.float32),
        scratch_types=[
            pltpu.VMEM((8, 2, _CK), jnp.int32),
            pltpu.VMEM((4, _CK, d), jnp.float32),
            pltpu.VMEM((4, _CK, d // 2), jnp.uint32),
            pltpu.VMEM_SHARED((n, d), jnp.float32),
        ] + [pltpu.SemaphoreType.DMA] * 24,
    )(_sc_agg_body)
    return kern(feat, ea, ids2, zeros)


def _node_mlp_body(x_ref, p_ref, wa_ref, ba_ref, wb_ref, bb_ref, o_ref):
    z = x_ref[...] + p_ref[0] + p_ref[1]
    z = jnp.maximum(
        jnp.dot(z, wa_ref[...], preferred_element_type=jnp.float32)
        + ba_ref[...], 0.0)
    z = jnp.dot(z, wb_ref[...], preferred_element_type=jnp.float32) + bb_ref[...]
    o_ref[...] = jnp.maximum(z, 0.0)


def _node_mlp(x, partials, Wa, ba, Wb, bb):
    n, d = x.shape
    BN = 2000
    assert n % BN == 0
    h = Wa.shape[1]
    full = lambda shape: pl.BlockSpec(shape, lambda i: (0, 0))
    return pl.pallas_call(
        _node_mlp_body,
        grid=(n // BN,),
        in_specs=[
            pl.BlockSpec((BN, d), lambda i: (i, 0)),
            pl.BlockSpec((2, BN, d), lambda i: (0, i, 0)),
            full((d, h)), full((1, h)),
            full((h, h)), full((1, h)),
        ],
        out_specs=pl.BlockSpec((BN, h), lambda i: (i, 0)),
        out_shape=jax.ShapeDtypeStruct((n, h), jnp.float32),
    )(x, partials, Wa, ba.reshape(1, -1), Wb, bb.reshape(1, -1))


def _final_body(h_ref, p_ref, batch_ref, wa_ref, ba_ref, wb_ref, bb_ref,
                wo_ref, bo_ref, o_ref, sums, counts):
    i = pl.program_id(0)
    nsteps = pl.num_programs(0)
    g = sums.shape[0]

    @pl.when(i == 0)
    def _():
        sums[...] = jnp.zeros_like(sums)
        counts[...] = jnp.zeros_like(counts)

    z = h_ref[...] + p_ref[0] + p_ref[1]
    z = jnp.maximum(
        jnp.dot(z, wa_ref[...], preferred_element_type=jnp.float32)
        + ba_ref[...], 0.0)
    h2 = jnp.dot(z, wb_ref[...], preferred_element_type=jnp.float32) + bb_ref[...]

    bids = batch_ref[0, 0, :]
    gids = lax.broadcasted_iota(jnp.int32, (g, bids.shape[0]), 0)
    onehot = (gids == bids[None, :]).astype(jnp.float32)
    sums[...] += jnp.dot(onehot, h2, preferred_element_type=jnp.float32)
    counts[...] += jnp.sum(onehot, axis=1, keepdims=True)

    @pl.when(i == nsteps - 1)
    def _():
        pooled = sums[...] / jnp.maximum(counts[...], 1.0)
        o_ref[...] = (jnp.dot(pooled, wo_ref[...],
                              preferred_element_type=jnp.float32)
                      + bo_ref[...])


def _final(h, partials, batch, Wa, ba, Wb, bb, Wo, bo, G):
    n, d = h.shape
    BN = 2000
    nb = n // BN
    o = Wo.shape[1]
    batch3 = batch.reshape(nb, 1, BN)
    full = lambda shape: pl.BlockSpec(shape, lambda i: (0, 0))
    return pl.pallas_call(
        _final_body,
        grid=(nb,),
        in_specs=[
            pl.BlockSpec((BN, d), lambda i: (i, 0)),
            pl.BlockSpec((2, BN, d), lambda i: (0, i, 0)),
            pl.BlockSpec((1, 1, BN), lambda i: (i, 0, 0)),
            full((d, d)), full((1, d)),
            full((d, d)), full((1, d)),
            full((d, o)), full((1, o)),
        ],
        out_specs=pl.BlockSpec((G, o), lambda i: (0, 0)),
        out_shape=jax.ShapeDtypeStruct((G, o), jnp.float32),
        scratch_shapes=[
            pltpu.VMEM((G, d), jnp.float32),
            pltpu.VMEM((G, 1), jnp.float32),
        ],
    )(h, partials, batch3, Wa, ba.reshape(1, -1), Wb, bb.reshape(1, -1),
      Wo, bo.reshape(1, -1))


def kernel(x, edge_index, edge_attr, batch, W_e, b_e, W_c1, b_c1, W1a, b1a,
           W1b, b1b, W_c2, b_c2, W2a, b2a, W2b, b2b, W_out, b_out):
    tot = edge_index.shape[1] // _CK
    ids2 = jnp.stack([edge_index[0].reshape(tot, _CK),
                      edge_index[1].reshape(tot, _CK)], axis=1)
    zeros = jnp.zeros_like(x)
    G = 64

    ea1, ea2 = _edge_proj(edge_attr, W_e, b_e, W_c1, b_c1, W_c2, b_c2)
    p1 = _sc_aggregate(x, ea1, ids2, zeros)
    h = _node_mlp(x, p1, W1a, b1a, W1b, b1b)
    p2 = _sc_aggregate(h, ea2, ids2, zeros)
    return _final(h, p2, batch, W2a, b2a, W2b, b2b, W_out, b_out, G)


# speedup vs baseline: 1.0248x; 1.0248x over previous
import functools

import jax
import jax.numpy as jnp
from jax import lax
from jax.experimental import pallas as pl
from jax.experimental.pallas import tpu as pltpu
from jax.experimental.pallas import tpu_sc as plsc


def _sel_onehots(d):
    r = lax.broadcasted_iota(jnp.int32, (d, d // 2), 0)
    k = lax.broadcasted_iota(jnp.int32, (d, d // 2), 1)
    tgt = 32 * (k // 16) + (k % 16)
    sel_a = (r == tgt).astype(jnp.float32)
    sel_b = (r == tgt + 16).astype(jnp.float32)
    return sel_a, sel_b


def _edge_proj_body(eattr_ref, we_ref, be_ref, wc1_ref, bc1_ref, wc2_ref,
                    bc2_ref, o1_ref, o2_ref):
    eb = eattr_ref[...]
    sel_a, sel_b = _sel_onehots(we_ref.shape[0])
    we = we_ref[...]
    be = be_ref[...]
    for wc_ref, bc_ref, o_ref in ((wc1_ref, bc1_ref, o1_ref),
                                  (wc2_ref, bc2_ref, o2_ref)):
        wf = jnp.dot(we, wc_ref[...], preferred_element_type=jnp.float32)
        bf = jnp.dot(be, wc_ref[...],
                     preferred_element_type=jnp.float32) + bc_ref[...]
        wfa = jnp.dot(wf, sel_a, preferred_element_type=jnp.float32)
        wfb = jnp.dot(wf, sel_b, preferred_element_type=jnp.float32)
        bfa = jnp.dot(bf, sel_a, preferred_element_type=jnp.float32)
        bfb = jnp.dot(bf, sel_b, preferred_element_type=jnp.float32)
        a = jnp.dot(eb, wfa, preferred_element_type=jnp.float32) + bfa
        b = jnp.dot(eb, wfb, preferred_element_type=jnp.float32) + bfb
        au = lax.bitcast_convert_type(a, jnp.uint32)
        bu = lax.bitcast_convert_type(b, jnp.uint32)
        lo = (au + jnp.uint32(0x8000)) >> jnp.uint32(16)
        hi = (bu + jnp.uint32(0x8000)) & jnp.uint32(0xFFFF0000)
        o_ref[...] = lo | hi


def _edge_proj(edge_attr, W_e, b_e, W_c1, b_c1, W_c2, b_c2):
    E, D = edge_attr.shape
    H = W_e.shape[1]
    BE = 4000
    assert E % BE == 0
    grid = (E // BE,)
    full = lambda shape: pl.BlockSpec(shape, lambda i: (0, 0))
    return pl.pallas_call(
        _edge_proj_body,
        grid=grid,
        in_specs=[
            pl.BlockSpec((BE, D), lambda i: (i, 0)),
            full((D, H)), full((1, H)),
            full((H, D)), full((1, D)),
            full((H, D)), full((1, D)),
        ],
        out_specs=[
            pl.BlockSpec((BE, D // 2), lambda i: (i, 0)),
            pl.BlockSpec((BE, D // 2), lambda i: (i, 0)),
        ],
        out_shape=[
            jax.ShapeDtypeStruct((E, D // 2), jnp.uint32),
            jax.ShapeDtypeStruct((E, D // 2), jnp.uint32),
        ],
    )(edge_attr, W_e, b_e.reshape(1, -1), W_c1, b_c1.reshape(1, -1),
      W_c2, b_c2.reshape(1, -1))


_CK = 40
_NCH = 250


def _sc_agg_body(feat_hbm, ea_hbm, ids_hbm, out_hbm,
                 ids, rows, eav, accum,
                 si0, si1, si2, si3, si4, si5, si6, si7,
                 sg0, sg1, sg2, sg3,
                 se0, se1, se2, se3, ss0, ss1, ss2, ss3, sz):
    nc = 2
    ns = 16
    c = lax.axis_index("c")
    s = lax.axis_index("s")
    wid = s * nc + c
    si = (si0, si1, si2, si3, si4, si5, si6, si7)
    sg = (sg0, sg1, sg2, sg3)
    se = (se0, se1, se2, se3)
    ss = (ss0, ss1, ss2, ss3)
    nch = _NCH
    ck = _CK
    d = rows.shape[2]
    estart = wid * nch * ck
    cstart = wid * nch
    ids_bytes = 2 * ck * 4
    ea_bytes = ck * (d // 2) * 4
    row_bytes = ck * d * 4

    def zrow(r, carry):
        for g in range(8):
            rows[0, r, pl.ds(g * 16, 16)] = jnp.zeros((16,), jnp.float32)
        return carry

    lax.fori_loop(0, ck, zrow, 0)
    n_nodes = accum.shape[0]
    per15 = 16 * ck
    base = s * per15

    @pl.when(s < 15)
    def _():
        for k in range(16):
            pltpu.async_copy(rows.at[0], accum.at[pl.ds(base + k * ck, ck)],
                             sz)
        for k in range(16):
            pltpu.make_async_copy(rows.at[0], accum.at[pl.ds(0, ck)],
                                  sz).wait()

    @pl.when(s == 15)
    def _():
        rem = (n_nodes - 15 * per15) // ck
        for k in range(rem):
            pltpu.async_copy(rows.at[0], accum.at[pl.ds(base + k * ck, ck)],
                             sz)
        for k in range(rem):
            pltpu.make_async_copy(rows.at[0], accum.at[pl.ds(0, ck)],
                                  sz).wait()

    plsc.subcore_barrier()

    def fetch_ids(j, b):
        pltpu.async_copy(ids_hbm.at[cstart + j], ids.at[b], si[b])

    def wait_ids(b):
        pltpu.make_async_copy(ids_hbm.at[0], ids.at[b], si[b]).wait()

    def fetch_rows(j, b, ib):
        pltpu.async_copy(ea_hbm.at[pl.ds(estart + j * ck, ck)], eav.at[b],
                         se[b])
        pltpu.async_copy(feat_hbm.at[ids.at[ib, 0]], rows.at[b], sg[b])

    def wait_rows(b):
        pltpu.make_async_copy(ea_hbm.at[pl.ds(0, ck)], eav.at[b],
                              se[b]).wait()
        pltpu.make_async_copy(feat_hbm.at[ids.at[0, 0]], rows.at[b],
                              sg[b]).wait()

    def scatter(b, ib):
        pltpu.async_copy(rows.at[b], accum.at[ids.at[ib, 1]], ss[b], add=True)

    def wait_scatter(b):
        pltpu.make_async_copy(rows.at[b], accum.at[ids.at[0, 1]],
                              ss[b]).wait()

    def compute(b):
        shp = jnp.full((16,), 16, jnp.uint32)
        msk = jnp.full((16,), 0xFFFF0000, jnp.uint32)

        def row(r, carry):
            for g in range(4):
                ew = eav[b, r, pl.ds(g * 16, 16)]
                ea = lax.bitcast_convert_type(ew << shp, jnp.float32)
                eb = lax.bitcast_convert_type(ew & msk, jnp.float32)
                slo = pl.ds(g * 32, 16)
                shi = pl.ds(g * 32 + 16, 16)
                rows[b, r, slo] = jnp.maximum(rows[b, r, slo] + ea, 0.0)
                rows[b, r, shi] = jnp.maximum(rows[b, r, shi] + eb, 0.0)
            return carry

        lax.fori_loop(0, ck, row, 0)

    def step(j, b, ib):
        @pl.when(j >= 2)
        def _():
            wait_scatter((b + 2) % 4)

        @pl.when(j + 3 < nch)
        def _():
            fetch_ids(j + 3, (ib + 3) % 8)

        @pl.when(j + 2 < nch)
        def _():
            wait_ids((ib + 2) % 8)
            fetch_rows(j + 2, (b + 2) % 4, (ib + 2) % 8)

        wait_rows(b)
        compute(b)
        scatter(b, ib)

    fetch_ids(0, 0)
    fetch_ids(1, 1)
    fetch_ids(2, 2)
    wait_ids(0)
    fetch_rows(0, 0, 0)
    wait_ids(1)
    fetch_rows(1, 1, 1)

    def octet(t, carry):
        for u in range(8):
            step(8 * t + u, u % 4, u)
        return carry

    lax.fori_loop(0, nch // 8, octet, 0)
    for e in range(nch - nch % 8, nch):
        step(e, e % 4, e % 8)
    wait_scatter((nch - 2) % 4)
    wait_scatter((nch - 1) % 4)
    plsc.subcore_barrier()

    @pl.when(s == 0)
    def _():
        pltpu.sync_copy(accum, out_hbm.at[c])


def _sc_aggregate(feat, ea, ids2):
    n, d = feat.shape
    mesh = plsc.VectorSubcoreMesh(core_axis_name="c", subcore_axis_name="s")
    kern = functools.partial(
        pl.kernel,
        mesh=mesh,
        out_type=jax.ShapeDtypeStruct((2, n, d), jnp.float32),
        scratch_types=[
            pltpu.VMEM((8, 2, _CK), jnp.int32),
            pltpu.VMEM((4, _CK, d), jnp.float32),
            pltpu.VMEM((4, _CK, d // 2), jnp.uint32),
            pltpu.VMEM_SHARED((n, d), jnp.float32),
        ] + [pltpu.SemaphoreType.DMA] * 21,
    )(_sc_agg_body)
    return kern(feat, ea, ids2)


def _node_mlp_body(x_ref, p_ref, wa_ref, ba_ref, wb_ref, bb_ref, o_ref):
    z = x_ref[...] + p_ref[0] + p_ref[1]
    z = jnp.maximum(
        jnp.dot(z, wa_ref[...], preferred_element_type=jnp.float32)
        + ba_ref[...], 0.0)
    z = jnp.dot(z, wb_ref[...], preferred_element_type=jnp.float32) + bb_ref[...]
    o_ref[...] = jnp.maximum(z, 0.0)


def _node_mlp(x, partials, Wa, ba, Wb, bb):
    n, d = x.shape
    BN = 2000
    assert n % BN == 0
    h = Wa.shape[1]
    full = lambda shape: pl.BlockSpec(shape, lambda i: (0, 0))
    return pl.pallas_call(
        _node_mlp_body,
        grid=(n // BN,),
        in_specs=[
            pl.BlockSpec((BN, d), lambda i: (i, 0)),
            pl.BlockSpec((2, BN, d), lambda i: (0, i, 0)),
            full((d, h)), full((1, h)),
            full((h, h)), full((1, h)),
        ],
        out_specs=pl.BlockSpec((BN, h), lambda i: (i, 0)),
        out_shape=jax.ShapeDtypeStruct((n, h), jnp.float32),
    )(x, partials, Wa, ba.reshape(1, -1), Wb, bb.reshape(1, -1))


def _final_body(h_ref, p_ref, batch_ref, wa_ref, ba_ref, wb_ref, bb_ref,
                wo_ref, bo_ref, o_ref, sums, counts):
    i = pl.program_id(0)
    nsteps = pl.num_programs(0)
    g = sums.shape[0]

    @pl.when(i == 0)
    def _():
        sums[...] = jnp.zeros_like(sums)
        counts[...] = jnp.zeros_like(counts)

    z = h_ref[...] + p_ref[0] + p_ref[1]
    z = jnp.maximum(
        jnp.dot(z, wa_ref[...], preferred_element_type=jnp.float32)
        + ba_ref[...], 0.0)
    h2 = jnp.dot(z, wb_ref[...], preferred_element_type=jnp.float32) + bb_ref[...]

    bids = batch_ref[0, 0, :]
    gids = lax.broadcasted_iota(jnp.int32, (g, bids.shape[0]), 0)
    onehot = (gids == bids[None, :]).astype(jnp.float32)
    sums[...] += jnp.dot(onehot, h2, preferred_element_type=jnp.float32)
    counts[...] += jnp.sum(onehot, axis=1, keepdims=True)

    @pl.when(i == nsteps - 1)
    def _():
        pooled = sums[...] / jnp.maximum(counts[...], 1.0)
        o_ref[...] = (jnp.dot(pooled, wo_ref[...],
                              preferred_element_type=jnp.float32)
                      + bo_ref[...])


def _final(h, partials, batch, Wa, ba, Wb, bb, Wo, bo, G):
    n, d = h.shape
    BN = 2000
    nb = n // BN
    o = Wo.shape[1]
    batch3 = batch.reshape(nb, 1, BN)
    full = lambda shape: pl.BlockSpec(shape, lambda i: (0, 0))
    return pl.pallas_call(
        _final_body,
        grid=(nb,),
        in_specs=[
            pl.BlockSpec((BN, d), lambda i: (i, 0)),
            pl.BlockSpec((2, BN, d), lambda i: (0, i, 0)),
            pl.BlockSpec((1, 1, BN), lambda i: (i, 0, 0)),
            full((d, d)), full((1, d)),
            full((d, d)), full((1, d)),
            full((d, o)), full((1, o)),
        ],
        out_specs=pl.BlockSpec((G, o), lambda i: (0, 0)),
        out_shape=jax.ShapeDtypeStruct((G, o), jnp.float32),
        scratch_shapes=[
            pltpu.VMEM((G, d), jnp.float32),
            pltpu.VMEM((G, 1), jnp.float32),
        ],
    )(h, partials, batch3, Wa, ba.reshape(1, -1), Wb, bb.reshape(1, -1),
      Wo, bo.reshape(1, -1))


def kernel(x, edge_index, edge_attr, batch, W_e, b_e, W_c1, b_c1, W1a, b1a,
           W1b, b1b, W_c2, b_c2, W2a, b2a, W2b, b2b, W_out, b_out):
    tot = edge_index.shape[1] // _CK
    ids2 = jnp.stack([edge_index[0].reshape(tot, _CK),
                      edge_index[1].reshape(tot, _CK)], axis=1)
    G = 64

    ea1, ea2 = _edge_proj(edge_attr, W_e, b_e, W_c1, b_c1, W_c2, b_c2)
    p1 = _sc_aggregate(x, ea1, ids2)
    h = _node_mlp(x, p1, W1a, b1a, W1b, b1b)
    p2 = _sc_aggregate(h, ea2, ids2)
    return _final(h, p2, batch, W2a, b2a, W2b, b2b, W_out, b_out, G)
